# baseline (speedup 1.0000x reference)
import functools
import jax
import jax.numpy as jnp
from jax import lax
from jax.experimental import pallas as pl
from jax.experimental.pallas import tpu as pltpu
from jax.experimental.pallas import tpu_sc as plsc

N = 10000
NP = 10240
E = 320000
NG = 64
HIDDEN = 1024
NOUT = 128

NC = 2
NS = 16
SLAB = E // NS
EB = 2000
NBLK = SLAB // EB
RB = 64
CAP = 2048
BM = 1024


def _make_matmul(d_in, d_out, fuse_prologue):
    def body(*refs):
        if fuse_prologue:
            numer_ref, denom_ref, b_ref, w_ref, a_ref, h_ref, as_ref, ad_ref = refs
            den = denom_ref[...]
            x = jnp.where(den > 0.0, numer_ref[...] / den, 0.0) + b_ref[...]
            x = jnp.maximum(x, 0.0)
        else:
            x_ref, w_ref, a_ref, h_ref, as_ref, ad_ref = refs
            x = x_ref[...]
        h = jnp.dot(x, w_ref[...], preferred_element_type=jnp.float32)
        h_ref[...] = h
        asad = jnp.dot(h, a_ref[...], preferred_element_type=jnp.float32)
        as_ref[...] = asad[:, 0:1]
        ad_ref[...] = asad[:, 1:2]

    if fuse_prologue:
        in_specs = [
            pl.BlockSpec((BM, d_in), lambda i: (i, 0)),
            pl.BlockSpec((BM, 1), lambda i: (i, 0)),
            pl.BlockSpec((1, d_in), lambda i: (0, 0)),
            pl.BlockSpec((d_in, d_out), lambda i: (0, 0)),
            pl.BlockSpec((d_out, 2), lambda i: (0, 0)),
        ]
    else:
        in_specs = [
            pl.BlockSpec((BM, d_in), lambda i: (i, 0)),
            pl.BlockSpec((d_in, d_out), lambda i: (0, 0)),
            pl.BlockSpec((d_out, 2), lambda i: (0, 0)),
        ]
    return pl.pallas_call(
        body,
        grid=(NP // BM,),
        in_specs=in_specs,
        out_specs=[
            pl.BlockSpec((BM, d_out), lambda i: (i, 0)),
            pl.BlockSpec((BM, 1), lambda i: (i, 0)),
            pl.BlockSpec((BM, 1), lambda i: (i, 0)),
        ],
        out_shape=[
            jax.ShapeDtypeStruct((NP, d_out), jnp.float32),
            jax.ShapeDtypeStruct((NP, 1), jnp.float32),
            jax.ShapeDtypeStruct((NP, 1), jnp.float32),
        ],
    )


def _make_edge_kernel(d, n_chunk_per_core):
    CH = NP // (NC * n_chunk_per_core)
    PERT = CH // NS
    F = d // 128
    RB = 128 // F
    RBF = 128

    mesh = plsc.VectorSubcoreMesh(
        core_axis_name="c", subcore_axis_name="s", num_cores=NC, num_subcores=NS)

    @functools.partial(
        pl.kernel,
        mesh=mesh,
        compiler_params=pltpu.CompilerParams(needs_layout_passes=False),
        out_type=[
            jax.ShapeDtypeStruct((NP * F, 128), jnp.float32),
            jax.ShapeDtypeStruct((NP,), jnp.float32),
        ],
        scratch_types=[
            pltpu.VMEM((NP,), jnp.float32),
            pltpu.VMEM((NP,), jnp.float32),
            pltpu.VMEM((EB,), jnp.int32),
            pltpu.VMEM((EB,), jnp.int32),
            pltpu.VMEM((EB,), jnp.int32),
            pltpu.VMEM((EB,), jnp.int32),
            pltpu.VMEM((CAP,), jnp.int32),
            pltpu.VMEM((CAP,), jnp.int32),
            pltpu.VMEM((CAP,), jnp.float32),
            pltpu.VMEM((CAP * F,), jnp.int32),
            pltpu.VMEM((CAP * F,), jnp.int32),
            pltpu.VMEM((RBF,), jnp.int32),
            pltpu.VMEM((RBF,), jnp.int32),
            pltpu.VMEM((RB,), jnp.int32),
            pltpu.VMEM((RB,), jnp.int32),
            pltpu.VMEM((RB,), jnp.float32),
            pltpu.VMEM((RB,), jnp.float32),
            pltpu.VMEM((RBF, 128), jnp.float32),
            pltpu.VMEM((RBF, 128), jnp.float32),
            pltpu.VMEM((32, 128), jnp.float32),
            pltpu.VMEM((512,), jnp.float32),
            pltpu.VMEM((512,), jnp.float32),
            pltpu.SemaphoreType.DMA,
            pltpu.SemaphoreType.DMA,
            pltpu.SemaphoreType.DMA,
            pltpu.SemaphoreType.DMA,
            pltpu.VMEM_SHARED((CH * F, 128), jnp.float32),
            pltpu.VMEM_SHARED((CH,), jnp.float32),
        ],
    )
    def edge_kernel(src_hbm, dst_hbm, as_hbm, ad_hbm, h_hbm,
                    numer_hbm, denom_hbm,
                    as_v, ad_v, srcraw0, dstraw0, srcraw1, dstraw1,
                    cpk, cdstl, wbuf, srcsub, dstsub,
                    idxsub0, idxsub1, idxstage0, idxstage1,
                    wstage0, wstage1,
                    rowbuf, rowbuf1, zbuf, zden, dbuf,
                    gsem0, gsem1, bsem0, bsem1,
                    numer_s, denom_s):
        srawbufs = ((srcraw0, dstraw0), (srcraw1, dstraw1))
        bsems = (bsem0, bsem1)
        rowbufs = (rowbuf, rowbuf1)
        idxsubs = (idxsub0, idxsub1)
        idxstages = (idxstage0, idxstage1)
        wstages = (wstage0, wstage1)
        gsems = (gsem0, gsem1)
        c = lax.axis_index("c")
        s = lax.axis_index("s")

        pltpu.sync_copy(as_hbm, as_v)
        pltpu.sync_copy(ad_hbm, ad_v)

        zv = jnp.zeros((16,), jnp.float32)

        def zb_body(i, _):
            r = i // 8
            g = lax.rem(i, 8)
            zbuf[r, pl.ds(g * 16, 16)] = zv
            return 0
        lax.fori_loop(0, 32 * 8, zb_body, 0)

        def zd_body(i, _):
            zden[pl.ds(i * 16, 16)] = zv
            return 0
        lax.fori_loop(0, 512 // 16, zd_body, 0)

        ninf = jnp.full((16,), -jnp.inf, jnp.float32)

        def mx_body(i, carry):
            va, vb = carry
            va = jnp.maximum(va, as_v[pl.ds(i * 16, 16)])
            vb = jnp.maximum(vb, ad_v[pl.ds(i * 16, 16)])
            return va, vb
        vmax_as, vmax_ad = lax.fori_loop(0, NP // 16, mx_body, (ninf, ninf))

        def lane_max(v):
            for sh in (1, 2, 4, 8):
                idx = lax.rem(lax.iota(jnp.int32, 16) + sh, 16)
                v = jnp.maximum(v, v.at[idx].get(mode="promise_in_bounds"))
            return v[0]

        mtot = lane_max(vmax_as) + lane_max(vmax_ad)
        mg = jnp.maximum(mtot, 0.2 * mtot)

        for q in range(n_chunk_per_core):
            lo = (c * n_chunk_per_core + q) * CH

            def znum(t, _):
                pltpu.sync_copy(
                    zbuf, numer_s.at[pl.ds(s * PERT * F + t * 32, 32)])
                return 0
            lax.fori_loop(0, PERT * F // 32, znum, 0)
            pltpu.sync_copy(zden.at[pl.ds(0, PERT)],
                            denom_s.at[pl.ds(s * PERT, PERT)])
            plsc.subcore_barrier()

            lanes = lax.iota(jnp.int32, 16)

            def issue_block(j, pb):
                base = s * SLAB + j * EB
                pltpu.async_copy(src_hbm.at[pl.ds(base, EB)],
                                 srawbufs[pb][0], bsems[pb])
                pltpu.async_copy(dst_hbm.at[pl.ds(base, EB)],
                                 srawbufs[pb][1], bsems[pb])

            issue_block(jnp.int32(0), 0)

            def blk(j, pb):
                srcraw, dstraw = srawbufs[pb]
                base = s * SLAB + j * EB
                pltpu.make_async_copy(src_hbm.at[pl.ds(base, EB)],
                                      srcraw, bsems[pb]).wait()
                pltpu.make_async_copy(dst_hbm.at[pl.ds(base, EB)],
                                      dstraw, bsems[pb]).wait()

                @pl.when(j + 1 < NBLK)
                def _():
                    issue_block(j + 1, (pb + 1) % 2)

                def filt(v, cnt):
                    vs = srcraw[pl.ds(v * 16, 16)]
                    vd = dstraw[pl.ds(v * 16, 16)]
                    inb = (vd >= lo) & (vd < lo + CH)
                    key = jnp.where(inb, lanes, lanes + 16)
                    packed = vd * 16384 + vs
                    _, pk = lax.sort((key, packed), num_keys=1)
                    cpk[pl.ds(cnt, 16)] = pk
                    return cnt + plsc.all_reduce_population_count(inb)[0]
                cnt = lax.fori_loop(0, EB // 16, filt, jnp.int32(0),
                                    unroll=2)

                nb = (cnt + (RB - 1)) // RB

                def wloop(v, _):
                    valid = (v * 16 + lanes) < cnt
                    pk = cpk[pl.ds(v * 16, 16)]
                    vs = jnp.where(valid, lax.rem(pk, 16384), 0)
                    vd = jnp.where(valid, pk // 16384, lo)
                    a1 = plsc.load_gather(as_v, [vs])
                    a2 = plsc.load_gather(ad_v, [vd])
                    e = a1 + a2
                    e = jnp.maximum(e, 0.2 * e)
                    w = jnp.where(valid, jnp.exp(e - mg), 0.0)
                    wbuf[pl.ds(v * 16, 16)] = w
                    vdl = vd - lo
                    cdstl[pl.ds(v * 16, 16)] = vdl
                    pos0 = (v * 16 + lanes) * F
                    for k in range(F):
                        plsc.store_scatter(srcsub, [pos0 + k], vs * F + k)
                        plsc.store_scatter(dstsub, [pos0 + k], vdl * F + k)
                    return 0
                lax.fori_loop(0, nb * (RB // 16), wloop, 0)

                def issue_g(rb, b):
                    pltpu.async_copy(
                        h_hbm.at[srcsub.at[pl.ds(rb * RBF, RBF)]],
                        rowbufs[b], gsems[b])

                for b in range(2):
                    @pl.when(b < nb)
                    def _(b=b):
                        issue_g(jnp.int32(b), b)

                def ring(p, _):
                    for b in range(2):
                        rb = p * 2 + b

                        @pl.when(rb < nb)
                        def _(rb=rb, b=b):
                            buf = rowbufs[b]
                            pltpu.make_async_copy(
                                h_hbm.at[srcsub.at[pl.ds(rb * RBF, RBF)]],
                                buf, gsems[b]).wait()

                            def cpy(k2, _):
                                idxsubs[b][pl.ds(k2 * 16, 16)] = (
                                    dstsub[pl.ds(rb * RBF + k2 * 16, 16)])
                                return 0
                            lax.fori_loop(0, RBF // 16, cpy, 0)

                            def cpy2(k2, _):
                                idxstages[b][pl.ds(k2 * 16, 16)] = (
                                    cdstl[pl.ds(rb * RB + k2 * 16, 16)])
                                wstages[b][pl.ds(k2 * 16, 16)] = (
                                    wbuf[pl.ds(rb * RB + k2 * 16, 16)])
                                return 0
                            lax.fori_loop(0, RB // 16, cpy2, 0)

                            def scale(r, _):
                                wv = plsc.load_gather(
                                    wstages[b],
                                    [jnp.full((16,), r // F, jnp.int32)])
                                for g in range(8):
                                    buf[r, pl.ds(g * 16, 16)] = (
                                        buf[r, pl.ds(g * 16, 16)] * wv)
                                return 0
                            lax.fori_loop(0, RBF, scale, 0, unroll=2)

                            pltpu.sync_copy(buf, numer_s.at[idxsubs[b]],
                                            add=True)
                            pltpu.sync_copy(wstages[b],
                                            denom_s.at[idxstages[b]],
                                            add=True)

                            @pl.when(rb + 2 < nb)
                            def _():
                                issue_g(rb + 2, b)
                    return 0
                lax.fori_loop(0, (nb + 1) // 2, ring, 0)
                return 0

            def bpair(p2, _):
                for pb in range(2):
                    blk(p2 * 2 + pb, pb)
                return 0
            lax.fori_loop(0, NBLK // 2, bpair, 0)
            plsc.subcore_barrier()

            def wb(t, _):
                r0 = s * PERT * F + t * 32
                pltpu.sync_copy(numer_s.at[pl.ds(r0, 32)],
                                rowbuf.at[pl.ds(0, 32)])
                pltpu.sync_copy(rowbuf.at[pl.ds(0, 32)],
                                numer_hbm.at[pl.ds(lo * F + r0, 32)])
                return 0
            lax.fori_loop(0, PERT * F // 32, wb, 0)
            pltpu.sync_copy(denom_s.at[pl.ds(s * PERT, PERT)],
                            dbuf.at[pl.ds(0, PERT)])
            pltpu.sync_copy(dbuf.at[pl.ds(0, PERT)],
                            denom_hbm.at[pl.ds(lo + s * PERT, PERT)])
            plsc.subcore_barrier()

    return edge_kernel


def _make_pool_mlp():
    def body(numer_ref, denom_ref, b_ref, pb_ref, w1_ref, b1_ref,
             w2_ref, b2_ref, out_ref, pooled, counts):
        i = pl.program_id(0)

        @pl.when(i == 0)
        def _():
            pooled[...] = jnp.zeros_like(pooled)
            counts[...] = jnp.zeros_like(counts)

        den = denom_ref[...]
        x = jnp.where(den > 0.0, numer_ref[...] / den, 0.0) + b_ref[...]
        x = jnp.maximum(x, 0.0)
        pb = pb_ref[0, 0, :]
        oh = (pb[None, :] == lax.broadcasted_iota(jnp.int32, (NG, BM), 0)
              ).astype(jnp.float32)
        pooled[...] += jnp.dot(oh, x, preferred_element_type=jnp.float32)
        counts[...] += jnp.sum(oh, axis=1, keepdims=True)

        @pl.when(i == NP // BM - 1)
        def _():
            gc = pooled[...] / jnp.maximum(counts[...], 1.0)
            z = jnp.dot(gc, w1_ref[...], preferred_element_type=jnp.float32)
            z = jnp.maximum(z + b1_ref[...], 0.0)
            out_ref[...] = (jnp.dot(z, w2_ref[...],
                                    preferred_element_type=jnp.float32)
                            + b2_ref[...])

    return pl.pallas_call(
        body,
        grid=(NP // BM,),
        in_specs=[
            pl.BlockSpec((BM, 4 * 128), lambda i: (i, 0)),
            pl.BlockSpec((BM, 1), lambda i: (i, 0)),
            pl.BlockSpec((1, 4 * 128), lambda i: (0, 0)),
            pl.BlockSpec((1, 1, BM), lambda i: (i, 0, 0)),
            pl.BlockSpec((4 * 128, HIDDEN), lambda i: (0, 0)),
            pl.BlockSpec((1, HIDDEN), lambda i: (0, 0)),
            pl.BlockSpec((HIDDEN, NOUT), lambda i: (0, 0)),
            pl.BlockSpec((1, NOUT), lambda i: (0, 0)),
        ],
        out_specs=pl.BlockSpec((NG, NOUT), lambda i: (0, 0)),
        out_shape=jax.ShapeDtypeStruct((NG, NOUT), jnp.float32),
        scratch_shapes=[
            pltpu.VMEM((NG, 4 * 128), jnp.float32),
            pltpu.VMEM((NG, 1), jnp.float32),
        ],
    )


_mm1 = _make_matmul(128, 128, False)
_mm2 = _make_matmul(128, 256, True)
_mm3 = _make_matmul(256, 512, True)
_pool_mlp = _make_pool_mlp()

_edge_cache = {}


def _get_edge(d, n_chunk_per_core):
    key = (d, n_chunk_per_core)
    if key not in _edge_cache:
        _edge_cache[key] = _make_edge_kernel(d, n_chunk_per_core)
    return _edge_cache[key]


@jax.jit
def _run(feature, edge_index, protein_batch, W1, a_src1, a_dst1, b1,
         W2, a_src2, a_dst2, b2, W3, a_src3, a_dst3, b3,
         Wfc1, bfc1, Wfc2, bfc2):
    src = edge_index[0].astype(jnp.int32)
    dst = edge_index[1].astype(jnp.int32)
    pb = jnp.pad(protein_batch.astype(jnp.int32), (0, NP - N),
                 constant_values=NG)
    pb3d = pb.reshape(NP // BM, 1, BM)
    x0 = jnp.pad(feature, ((0, NP - N), (0, 0)))
    A1 = jnp.stack([a_src1, a_dst1], axis=1)
    A2 = jnp.stack([a_src2, a_dst2], axis=1)
    A3 = jnp.stack([a_src3, a_dst3], axis=1)

    h1, as1, ad1 = _mm1(x0, W1, A1)
    num1, den1 = _get_edge(128, 2)(src, dst, as1.reshape(NP), ad1.reshape(NP),
                                   h1.reshape(-1, 128))
    h2, as2, ad2 = _mm2(num1.reshape(NP, 128), den1.reshape(NP, 1),
                        b1.reshape(1, -1), W2, A2)
    num2, den2 = _get_edge(256, 2)(src, dst, as2.reshape(NP), ad2.reshape(NP),
                                   h2.reshape(-1, 128))
    h3, as3, ad3 = _mm3(num2.reshape(NP, 256), den2.reshape(NP, 1),
                        b2.reshape(1, -1), W3, A3)
    num3, den3 = _get_edge(512, 5)(src, dst, as3.reshape(NP), ad3.reshape(NP),
                                   h3.reshape(-1, 128))
    return _pool_mlp(num3.reshape(NP, 512), den3.reshape(NP, 1),
                     b3.reshape(1, -1), pb3d,
                     Wfc1, bfc1.reshape(1, -1), Wfc2, bfc2.reshape(1, -1))


def kernel(feature, edge_index, protein_batch, W1, a_src1, a_dst1, b1,
           W2, a_src2, a_dst2, b2, W3, a_src3, a_dst3, b3,
           Wfc1, bfc1, Wfc2, bfc2):
    return _run(feature, edge_index, protein_batch, W1, a_src1, a_dst1, b1,
                W2, a_src2, a_dst2, b2, W3, a_src3, a_dst3, b3,
                Wfc1, bfc1, Wfc2, bfc2)


# speedup vs baseline: 1.2685x; 1.2685x over previous
import functools
import jax
import jax.numpy as jnp
from jax import lax
from jax.experimental import pallas as pl
from jax.experimental.pallas import tpu as pltpu
from jax.experimental.pallas import tpu_sc as plsc

N = 10000
NP = 10240
E = 320000
NG = 64
HIDDEN = 1024
NOUT = 128

NC = 2
NS = 16
SLAB = E // NS
EB = 2000
NBLK = SLAB // EB
RB = 64
CAP = 2048
BM = 1024


def _make_matmul(d_in, d_out, fuse_prologue):
    def body(*refs):
        if fuse_prologue:
            numer_ref, denom_ref, b_ref, w_ref, a_ref, h_ref, as_ref, ad_ref = refs
            den = denom_ref[...]
            x = jnp.where(den > 0.0, numer_ref[...] / den, 0.0) + b_ref[...]
            x = jnp.maximum(x, 0.0)
        else:
            x_ref, w_ref, a_ref, h_ref, as_ref, ad_ref = refs
            x = x_ref[...]
        h = jnp.dot(x, w_ref[...], preferred_element_type=jnp.float32)
        h_ref[...] = h
        asad = jnp.dot(h, a_ref[...], preferred_element_type=jnp.float32)
        as_ref[...] = asad[:, 0:1]
        ad_ref[...] = asad[:, 1:2]

    if fuse_prologue:
        in_specs = [
            pl.BlockSpec((BM, d_in), lambda i: (i, 0)),
            pl.BlockSpec((BM, 1), lambda i: (i, 0)),
            pl.BlockSpec((1, d_in), lambda i: (0, 0)),
            pl.BlockSpec((d_in, d_out), lambda i: (0, 0)),
            pl.BlockSpec((d_out, 2), lambda i: (0, 0)),
        ]
    else:
        in_specs = [
            pl.BlockSpec((BM, d_in), lambda i: (i, 0)),
            pl.BlockSpec((d_in, d_out), lambda i: (0, 0)),
            pl.BlockSpec((d_out, 2), lambda i: (0, 0)),
        ]
    return pl.pallas_call(
        body,
        grid=(NP // BM,),
        in_specs=in_specs,
        out_specs=[
            pl.BlockSpec((BM, d_out), lambda i: (i, 0)),
            pl.BlockSpec((BM, 1), lambda i: (i, 0)),
            pl.BlockSpec((BM, 1), lambda i: (i, 0)),
        ],
        out_shape=[
            jax.ShapeDtypeStruct((NP, d_out), jnp.float32),
            jax.ShapeDtypeStruct((NP, 1), jnp.float32),
            jax.ShapeDtypeStruct((NP, 1), jnp.float32),
        ],
    )


def _make_edge_kernel(d, n_chunk_per_core):
    CH = NP // (NC * n_chunk_per_core)
    PERT = CH // NS
    F = d // 128
    RB = 128 // F
    RBF = 128

    mesh = plsc.VectorSubcoreMesh(
        core_axis_name="c", subcore_axis_name="s", num_cores=NC, num_subcores=NS)

    @functools.partial(
        pl.kernel,
        mesh=mesh,
        compiler_params=pltpu.CompilerParams(needs_layout_passes=False),
        out_type=[
            jax.ShapeDtypeStruct((NP * F, 128), jnp.float32),
            jax.ShapeDtypeStruct((NP,), jnp.float32),
        ],
        scratch_types=[
            pltpu.VMEM((NP,), jnp.float32),
            pltpu.VMEM((NP,), jnp.float32),
            pltpu.VMEM((EB,), jnp.int32),
            pltpu.VMEM((EB,), jnp.int32),
            pltpu.VMEM((EB,), jnp.int32),
            pltpu.VMEM((EB,), jnp.int32),
            pltpu.VMEM((CAP,), jnp.int32),
            pltpu.VMEM((CAP,), jnp.int32),
            pltpu.VMEM((CAP,), jnp.float32),
            pltpu.VMEM((CAP * F,), jnp.int32),
            pltpu.VMEM((CAP * F,), jnp.int32),
            pltpu.VMEM((RBF,), jnp.int32),
            pltpu.VMEM((RBF,), jnp.int32),
            pltpu.VMEM((RB,), jnp.int32),
            pltpu.VMEM((RB,), jnp.int32),
            pltpu.VMEM((RB,), jnp.float32),
            pltpu.VMEM((RB,), jnp.float32),
            pltpu.VMEM((RBF, 128), jnp.float32),
            pltpu.VMEM((RBF, 128), jnp.float32),
            pltpu.VMEM((32, 128), jnp.float32),
            pltpu.VMEM((512,), jnp.float32),
            pltpu.VMEM((512,), jnp.float32),
            pltpu.SemaphoreType.DMA,
            pltpu.SemaphoreType.DMA,
            pltpu.SemaphoreType.DMA,
            pltpu.SemaphoreType.DMA,
            pltpu.VMEM_SHARED((CH * F, 128), jnp.float32),
            pltpu.VMEM_SHARED((CH,), jnp.float32),
        ],
    )
    def edge_kernel(src_hbm, dst_hbm, as_hbm, ad_hbm, h_hbm,
                    numer_hbm, denom_hbm,
                    as_v, ad_v, srcraw0, dstraw0, srcraw1, dstraw1,
                    cpk, cdstl, wbuf, srcsub, dstsub,
                    idxsub0, idxsub1, idxstage0, idxstage1,
                    wstage0, wstage1,
                    rowbuf, rowbuf1, zbuf, zden, dbuf,
                    gsem0, gsem1, bsem0, bsem1,
                    numer_s, denom_s):
        srawbufs = ((srcraw0, dstraw0), (srcraw1, dstraw1))
        bsems = (bsem0, bsem1)
        rowbufs = (rowbuf, rowbuf1)
        idxsubs = (idxsub0, idxsub1)
        idxstages = (idxstage0, idxstage1)
        wstages = (wstage0, wstage1)
        gsems = (gsem0, gsem1)
        c = lax.axis_index("c")
        s = lax.axis_index("s")

        pltpu.sync_copy(as_hbm, as_v)
        pltpu.sync_copy(ad_hbm, ad_v)

        zv = jnp.zeros((16,), jnp.float32)

        def zb_body(i, _):
            r = i // 8
            g = lax.rem(i, 8)
            zbuf[r, pl.ds(g * 16, 16)] = zv
            return 0
        lax.fori_loop(0, 32 * 8, zb_body, 0)

        def zd_body(i, _):
            zden[pl.ds(i * 16, 16)] = zv
            return 0
        lax.fori_loop(0, 512 // 16, zd_body, 0)

        ninf = jnp.full((16,), -jnp.inf, jnp.float32)

        def mx_body(i, carry):
            va, vb = carry
            va = jnp.maximum(va, as_v[pl.ds(i * 16, 16)])
            vb = jnp.maximum(vb, ad_v[pl.ds(i * 16, 16)])
            return va, vb
        vmax_as, vmax_ad = lax.fori_loop(0, NP // 16, mx_body, (ninf, ninf))

        def lane_max(v):
            for sh in (1, 2, 4, 8):
                idx = lax.rem(lax.iota(jnp.int32, 16) + sh, 16)
                v = jnp.maximum(v, v.at[idx].get(mode="promise_in_bounds"))
            return v[0]

        mtot = lane_max(vmax_as) + lane_max(vmax_ad)
        mg = jnp.maximum(mtot, 0.2 * mtot)

        for q in range(n_chunk_per_core):
            lo = (c * n_chunk_per_core + q) * CH

            def znum(t, _):
                pltpu.sync_copy(
                    zbuf, numer_s.at[pl.ds(s * PERT * F + t * 32, 32)])
                return 0
            lax.fori_loop(0, PERT * F // 32, znum, 0)
            pltpu.sync_copy(zden.at[pl.ds(0, PERT)],
                            denom_s.at[pl.ds(s * PERT, PERT)])
            plsc.subcore_barrier()

            lanes = lax.iota(jnp.int32, 16)

            def issue_block(j, pb):
                base = s * SLAB + j * EB
                pltpu.async_copy(src_hbm.at[pl.ds(base, EB)],
                                 srawbufs[pb][0], bsems[pb])
                pltpu.async_copy(dst_hbm.at[pl.ds(base, EB)],
                                 srawbufs[pb][1], bsems[pb])

            issue_block(jnp.int32(0), 0)

            def blk(j, pb):
                srcraw, dstraw = srawbufs[pb]
                base = s * SLAB + j * EB
                pltpu.make_async_copy(src_hbm.at[pl.ds(base, EB)],
                                      srcraw, bsems[pb]).wait()
                pltpu.make_async_copy(dst_hbm.at[pl.ds(base, EB)],
                                      dstraw, bsems[pb]).wait()

                @pl.when(j + 1 < NBLK)
                def _():
                    issue_block(j + 1, (pb + 1) % 2)

                def filt(v, cnt):
                    vs = srcraw[pl.ds(v * 16, 16)]
                    vd = dstraw[pl.ds(v * 16, 16)]
                    inb = (vd >= lo) & (vd < lo + CH)
                    key = jnp.where(inb, lanes, lanes + 16)
                    packed = vd * 16384 + vs
                    _, pk = lax.sort((key, packed), num_keys=1)
                    cpk[pl.ds(cnt, 16)] = pk
                    return cnt + plsc.all_reduce_population_count(inb)[0]
                cnt = lax.fori_loop(0, EB // 16, filt, jnp.int32(0),
                                    unroll=2)

                nb = (cnt + (RB - 1)) // RB

                def wloop(v, _):
                    valid = (v * 16 + lanes) < cnt
                    pk = cpk[pl.ds(v * 16, 16)]
                    vs = jnp.where(valid, lax.rem(pk, 16384), 0)
                    vd = jnp.where(valid, pk // 16384, lo)
                    a1 = plsc.load_gather(as_v, [vs])
                    a2 = plsc.load_gather(ad_v, [vd])
                    e = a1 + a2
                    e = jnp.maximum(e, 0.2 * e)
                    w = jnp.where(valid, jnp.exp(e - mg), 0.0)
                    wbuf[pl.ds(v * 16, 16)] = w
                    vdl = vd - lo
                    cdstl[pl.ds(v * 16, 16)] = vdl
                    pos0 = (v * 16 + lanes) * F
                    for k in range(F):
                        plsc.store_scatter(srcsub, [pos0 + k], vs * F + k)
                        plsc.store_scatter(dstsub, [pos0 + k], vdl * F + k)
                    return 0
                lax.fori_loop(0, nb * (RB // 16), wloop, 0)

                def issue_g(rb, b):
                    pltpu.async_copy(
                        h_hbm.at[srcsub.at[pl.ds(rb * RBF, RBF)]],
                        rowbufs[b], gsems[b])

                for b in range(2):
                    @pl.when(b < nb)
                    def _(b=b):
                        issue_g(jnp.int32(b), b)

                def ring(p, _):
                    for b in range(2):
                        rb = p * 2 + b

                        @pl.when(rb < nb)
                        def _(rb=rb, b=b):
                            buf = rowbufs[b]
                            pltpu.make_async_copy(
                                h_hbm.at[srcsub.at[pl.ds(rb * RBF, RBF)]],
                                buf, gsems[b]).wait()

                            def cpy(k2, _):
                                idxsubs[b][pl.ds(k2 * 16, 16)] = (
                                    dstsub[pl.ds(rb * RBF + k2 * 16, 16)])
                                return 0
                            lax.fori_loop(0, RBF // 16, cpy, 0)

                            def cpy2(k2, _):
                                idxstages[b][pl.ds(k2 * 16, 16)] = (
                                    cdstl[pl.ds(rb * RB + k2 * 16, 16)])
                                wstages[b][pl.ds(k2 * 16, 16)] = (
                                    wbuf[pl.ds(rb * RB + k2 * 16, 16)])
                                return 0
                            lax.fori_loop(0, RB // 16, cpy2, 0)

                            def scale(r, _):
                                wv = plsc.load_gather(
                                    wstages[b],
                                    [jnp.full((16,), r // F, jnp.int32)])
                                for g in range(8):
                                    buf[r, pl.ds(g * 16, 16)] = (
                                        buf[r, pl.ds(g * 16, 16)] * wv)
                                return 0
                            lax.fori_loop(0, RBF, scale, 0, unroll=2)

                            pltpu.sync_copy(buf, numer_s.at[idxsubs[b]],
                                            add=True)
                            pltpu.sync_copy(wstages[b],
                                            denom_s.at[idxstages[b]],
                                            add=True)

                            @pl.when(rb + 2 < nb)
                            def _():
                                issue_g(rb + 2, b)
                    return 0
                lax.fori_loop(0, (nb + 1) // 2, ring, 0)
                return 0

            def bpair(p2, _):
                for pb in range(2):
                    blk(p2 * 2 + pb, pb)
                return 0
            lax.fori_loop(0, NBLK // 2, bpair, 0)
            plsc.subcore_barrier()

            def wb(t, _):
                r0 = s * PERT * F + t * 32
                pltpu.sync_copy(numer_s.at[pl.ds(r0, 32)],
                                rowbuf.at[pl.ds(0, 32)])
                pltpu.sync_copy(rowbuf.at[pl.ds(0, 32)],
                                numer_hbm.at[pl.ds(lo * F + r0, 32)])
                return 0
            lax.fori_loop(0, PERT * F // 32, wb, 0)
            pltpu.sync_copy(denom_s.at[pl.ds(s * PERT, PERT)],
                            dbuf.at[pl.ds(0, PERT)])
            pltpu.sync_copy(dbuf.at[pl.ds(0, PERT)],
                            denom_hbm.at[pl.ds(lo + s * PERT, PERT)])
            plsc.subcore_barrier()

    return edge_kernel


def _make_pool_mlp():
    def body(numer_ref, denom_ref, b_ref, pb_ref, w1_ref, b1_ref,
             w2_ref, b2_ref, out_ref, pooled, counts):
        i = pl.program_id(0)

        @pl.when(i == 0)
        def _():
            pooled[...] = jnp.zeros_like(pooled)
            counts[...] = jnp.zeros_like(counts)

        den = denom_ref[...]
        x = jnp.where(den > 0.0, numer_ref[...] / den, 0.0) + b_ref[...]
        x = jnp.maximum(x, 0.0)
        pb = pb_ref[0, 0, :]
        oh = (pb[None, :] == lax.broadcasted_iota(jnp.int32, (NG, BM), 0)
              ).astype(jnp.float32)
        pooled[...] += jnp.dot(oh, x, preferred_element_type=jnp.float32)
        counts[...] += jnp.sum(oh, axis=1, keepdims=True)

        @pl.when(i == NP // BM - 1)
        def _():
            gc = pooled[...] / jnp.maximum(counts[...], 1.0)
            z = jnp.dot(gc, w1_ref[...], preferred_element_type=jnp.float32)
            z = jnp.maximum(z + b1_ref[...], 0.0)
            out_ref[...] = (jnp.dot(z, w2_ref[...],
                                    preferred_element_type=jnp.float32)
                            + b2_ref[...])

    return pl.pallas_call(
        body,
        grid=(NP // BM,),
        in_specs=[
            pl.BlockSpec((BM, 4 * 128), lambda i: (i, 0)),
            pl.BlockSpec((BM, 1), lambda i: (i, 0)),
            pl.BlockSpec((1, 4 * 128), lambda i: (0, 0)),
            pl.BlockSpec((1, 1, BM), lambda i: (i, 0, 0)),
            pl.BlockSpec((4 * 128, HIDDEN), lambda i: (0, 0)),
            pl.BlockSpec((1, HIDDEN), lambda i: (0, 0)),
            pl.BlockSpec((HIDDEN, NOUT), lambda i: (0, 0)),
            pl.BlockSpec((1, NOUT), lambda i: (0, 0)),
        ],
        out_specs=pl.BlockSpec((NG, NOUT), lambda i: (0, 0)),
        out_shape=jax.ShapeDtypeStruct((NG, NOUT), jnp.float32),
        scratch_shapes=[
            pltpu.VMEM((NG, 4 * 128), jnp.float32),
            pltpu.VMEM((NG, 1), jnp.float32),
        ],
    )


_mm1 = _make_matmul(128, 128, False)
_mm2 = _make_matmul(128, 256, True)
_mm3 = _make_matmul(256, 512, True)
_pool_mlp = _make_pool_mlp()

_edge_cache = {}


def _get_edge(d, n_chunk_per_core):
    key = (d, n_chunk_per_core)
    if key not in _edge_cache:
        _edge_cache[key] = _make_edge_kernel(d, n_chunk_per_core)
    return _edge_cache[key]


@jax.jit
def _run(feature, edge_index, protein_batch, W1, a_src1, a_dst1, b1,
         W2, a_src2, a_dst2, b2, W3, a_src3, a_dst3, b3,
         Wfc1, bfc1, Wfc2, bfc2):
    src = edge_index[0].astype(jnp.int32)
    dst = edge_index[1].astype(jnp.int32)
    pb = jnp.pad(protein_batch.astype(jnp.int32), (0, NP - N),
                 constant_values=NG)
    pb3d = pb.reshape(NP // BM, 1, BM)
    x0 = jnp.pad(feature, ((0, NP - N), (0, 0)))
    A1 = jnp.stack([a_src1, a_dst1], axis=1)
    A2 = jnp.stack([a_src2, a_dst2], axis=1)
    A3 = jnp.stack([a_src3, a_dst3], axis=1)

    h1, as1, ad1 = _mm1(x0, W1, A1)
    num1, den1 = _get_edge(128, 1)(src, dst, as1.reshape(NP), ad1.reshape(NP),
                                   h1.reshape(-1, 128))
    h2, as2, ad2 = _mm2(num1.reshape(NP, 128), den1.reshape(NP, 1),
                        b1.reshape(1, -1), W2, A2)
    num2, den2 = _get_edge(256, 2)(src, dst, as2.reshape(NP), ad2.reshape(NP),
                                   h2.reshape(-1, 128))
    h3, as3, ad3 = _mm3(num2.reshape(NP, 256), den2.reshape(NP, 1),
                        b2.reshape(1, -1), W3, A3)
    num3, den3 = _get_edge(512, 4)(src, dst, as3.reshape(NP), ad3.reshape(NP),
                                   h3.reshape(-1, 128))
    return _pool_mlp(num3.reshape(NP, 512), den3.reshape(NP, 1),
                     b3.reshape(1, -1), pb3d,
                     Wfc1, bfc1.reshape(1, -1), Wfc2, bfc2.reshape(1, -1))


def kernel(feature, edge_index, protein_batch, W1, a_src1, a_dst1, b1,
           W2, a_src2, a_dst2, b2, W3, a_src3, a_dst3, b3,
           Wfc1, bfc1, Wfc2, bfc2):
    return _run(feature, edge_index, protein_batch, W1, a_src1, a_dst1, b1,
                W2, a_src2, a_dst2, b2, W3, a_src3, a_dst3, b3,
                Wfc1, bfc1, Wfc2, bfc2)


# speedup vs baseline: 1.2701x; 1.0013x over previous
import functools
import jax
import jax.numpy as jnp
from jax import lax
from jax.experimental import pallas as pl
from jax.experimental.pallas import tpu as pltpu
from jax.experimental.pallas import tpu_sc as plsc

N = 10000
NP = 10240
E = 320000
NG = 64
HIDDEN = 1024
NOUT = 128

NC = 2
NS = 16
SLAB = E // NS
EB = 2000
NBLK = SLAB // EB
RB = 64
CAP = 2048
BM = 1024


def _make_matmul(d_in, d_out, fuse_prologue):
    def body(*refs):
        if fuse_prologue:
            numer_ref, denom_ref, b_ref, w_ref, a_ref, h_ref, as_ref, ad_ref = refs
            den = denom_ref[...]
            x = jnp.where(den > 0.0, numer_ref[...] / den, 0.0) + b_ref[...]
            x = jnp.maximum(x, 0.0)
        else:
            x_ref, w_ref, a_ref, h_ref, as_ref, ad_ref = refs
            x = x_ref[...]
        h = jnp.dot(x, w_ref[...], preferred_element_type=jnp.float32)
        h_ref[...] = h
        asad = jnp.dot(h, a_ref[...], preferred_element_type=jnp.float32)
        as_ref[...] = asad[:, 0:1]
        ad_ref[...] = asad[:, 1:2]

    if fuse_prologue:
        in_specs = [
            pl.BlockSpec((BM, d_in), lambda i: (i, 0)),
            pl.BlockSpec((BM, 1), lambda i: (i, 0)),
            pl.BlockSpec((1, d_in), lambda i: (0, 0)),
            pl.BlockSpec((d_in, d_out), lambda i: (0, 0)),
            pl.BlockSpec((d_out, 2), lambda i: (0, 0)),
        ]
    else:
        in_specs = [
            pl.BlockSpec((BM, d_in), lambda i: (i, 0)),
            pl.BlockSpec((d_in, d_out), lambda i: (0, 0)),
            pl.BlockSpec((d_out, 2), lambda i: (0, 0)),
        ]
    return pl.pallas_call(
        body,
        grid=(NP // BM,),
        in_specs=in_specs,
        out_specs=[
            pl.BlockSpec((BM, d_out), lambda i: (i, 0)),
            pl.BlockSpec((BM, 1), lambda i: (i, 0)),
            pl.BlockSpec((BM, 1), lambda i: (i, 0)),
        ],
        out_shape=[
            jax.ShapeDtypeStruct((NP, d_out), jnp.float32),
            jax.ShapeDtypeStruct((NP, 1), jnp.float32),
            jax.ShapeDtypeStruct((NP, 1), jnp.float32),
        ],
    )


def _make_edge_kernel(d, n_chunk_per_core):
    CH = NP // (NC * n_chunk_per_core)
    PERT = CH // NS
    F = d // 128
    RB = 128 // F
    RBF = 128

    mesh = plsc.VectorSubcoreMesh(
        core_axis_name="c", subcore_axis_name="s", num_cores=NC, num_subcores=NS)

    @functools.partial(
        pl.kernel,
        mesh=mesh,
        compiler_params=pltpu.CompilerParams(needs_layout_passes=False),
        out_type=[
            jax.ShapeDtypeStruct((NP * F, 128), jnp.float32),
            jax.ShapeDtypeStruct((NP,), jnp.float32),
        ],
        scratch_types=[
            pltpu.VMEM((NP,), jnp.float32),
            pltpu.VMEM((NP,), jnp.float32),
            pltpu.VMEM((EB,), jnp.int32),
            pltpu.VMEM((EB,), jnp.int32),
            pltpu.VMEM((EB,), jnp.int32),
            pltpu.VMEM((EB,), jnp.int32),
            pltpu.VMEM((CAP,), jnp.int32),
            pltpu.VMEM((CAP,), jnp.int32),
            pltpu.VMEM((CAP,), jnp.float32),
            pltpu.VMEM((CAP * F,), jnp.int32),
            pltpu.VMEM((CAP * F,), jnp.int32),
            pltpu.VMEM((RBF,), jnp.int32),
            pltpu.VMEM((RBF,), jnp.int32),
            pltpu.VMEM((RB,), jnp.int32),
            pltpu.VMEM((RB,), jnp.int32),
            pltpu.VMEM((RB,), jnp.float32),
            pltpu.VMEM((RB,), jnp.float32),
            pltpu.VMEM((RBF, 128), jnp.float32),
            pltpu.VMEM((RBF, 128), jnp.float32),
            pltpu.VMEM((32, 128), jnp.float32),
            pltpu.VMEM((512,), jnp.float32),
            pltpu.VMEM((512,), jnp.float32),
            pltpu.SemaphoreType.DMA,
            pltpu.SemaphoreType.DMA,
            pltpu.SemaphoreType.DMA,
            pltpu.SemaphoreType.DMA,
            pltpu.VMEM_SHARED((CH * F, 128), jnp.float32),
            pltpu.VMEM_SHARED((CH,), jnp.float32),
        ],
    )
    def edge_kernel(src_hbm, dst_hbm, as_hbm, ad_hbm, h_hbm,
                    numer_hbm, denom_hbm,
                    as_v, ad_v, srcraw0, dstraw0, srcraw1, dstraw1,
                    cpk, cdstl, wbuf, srcsub, dstsub,
                    idxsub0, idxsub1, idxstage0, idxstage1,
                    wstage0, wstage1,
                    rowbuf, rowbuf1, zbuf, zden, dbuf,
                    gsem0, gsem1, bsem0, bsem1,
                    numer_s, denom_s):
        srawbufs = ((srcraw0, dstraw0), (srcraw1, dstraw1))
        bsems = (bsem0, bsem1)
        rowbufs = (rowbuf, rowbuf1)
        idxsubs = (idxsub0, idxsub1)
        idxstages = (idxstage0, idxstage1)
        wstages = (wstage0, wstage1)
        gsems = (gsem0, gsem1)
        c = lax.axis_index("c")
        s = lax.axis_index("s")

        pltpu.sync_copy(as_hbm, as_v)
        pltpu.sync_copy(ad_hbm, ad_v)

        zv = jnp.zeros((16,), jnp.float32)

        def zb_body(i, _):
            r = i // 8
            g = lax.rem(i, 8)
            zbuf[r, pl.ds(g * 16, 16)] = zv
            return 0
        lax.fori_loop(0, 32 * 8, zb_body, 0)

        def zd_body(i, _):
            zden[pl.ds(i * 16, 16)] = zv
            return 0
        lax.fori_loop(0, 512 // 16, zd_body, 0)

        ninf = jnp.full((16,), -jnp.inf, jnp.float32)

        def mx_body(i, carry):
            va, vb = carry
            va = jnp.maximum(va, as_v[pl.ds(i * 16, 16)])
            vb = jnp.maximum(vb, ad_v[pl.ds(i * 16, 16)])
            return va, vb
        vmax_as, vmax_ad = lax.fori_loop(0, NP // 16, mx_body, (ninf, ninf))

        def lane_max(v):
            for sh in (1, 2, 4, 8):
                idx = lax.rem(lax.iota(jnp.int32, 16) + sh, 16)
                v = jnp.maximum(v, v.at[idx].get(mode="promise_in_bounds"))
            return v[0]

        mtot = lane_max(vmax_as) + lane_max(vmax_ad)
        mg = jnp.maximum(mtot, 0.2 * mtot)

        for q in range(n_chunk_per_core):
            lo = (c * n_chunk_per_core + q) * CH

            def znum(t, _):
                pltpu.sync_copy(
                    zbuf, numer_s.at[pl.ds(s * PERT * F + t * 32, 32)])
                return 0
            lax.fori_loop(0, PERT * F // 32, znum, 0)
            pltpu.sync_copy(zden.at[pl.ds(0, PERT)],
                            denom_s.at[pl.ds(s * PERT, PERT)])
            plsc.subcore_barrier()

            lanes = lax.iota(jnp.int32, 16)

            def issue_block(j, pb):
                base = s * SLAB + j * EB
                pltpu.async_copy(src_hbm.at[pl.ds(base, EB)],
                                 srawbufs[pb][0], bsems[pb])
                pltpu.async_copy(dst_hbm.at[pl.ds(base, EB)],
                                 srawbufs[pb][1], bsems[pb])

            issue_block(jnp.int32(0), 0)

            def blk(j, pb):
                srcraw, dstraw = srawbufs[pb]
                base = s * SLAB + j * EB
                pltpu.make_async_copy(src_hbm.at[pl.ds(base, EB)],
                                      srcraw, bsems[pb]).wait()
                pltpu.make_async_copy(dst_hbm.at[pl.ds(base, EB)],
                                      dstraw, bsems[pb]).wait()

                @pl.when(j + 1 < NBLK)
                def _():
                    issue_block(j + 1, (pb + 1) % 2)

                def filt(v, cnt):
                    vs = srcraw[pl.ds(v * 16, 16)]
                    vd = dstraw[pl.ds(v * 16, 16)]
                    inb = (vd >= lo) & (vd < lo + CH)
                    key = jnp.where(inb, lanes, lanes + 16)
                    packed = vd * 16384 + vs
                    _, pk = lax.sort((key, packed), num_keys=1)
                    cpk[pl.ds(cnt, 16)] = pk
                    return cnt + plsc.all_reduce_population_count(inb)[0]
                cnt = lax.fori_loop(0, EB // 16, filt, jnp.int32(0),
                                    unroll=2)

                nb = (cnt + (RB - 1)) // RB

                def wloop(v, _):
                    valid = (v * 16 + lanes) < cnt
                    pk = cpk[pl.ds(v * 16, 16)]
                    vs = jnp.where(valid, lax.rem(pk, 16384), 0)
                    vd = jnp.where(valid, pk // 16384, lo)
                    a1 = plsc.load_gather(as_v, [vs])
                    a2 = plsc.load_gather(ad_v, [vd])
                    e = a1 + a2
                    e = jnp.maximum(e, 0.2 * e)
                    w = jnp.where(valid, jnp.exp(e - mg), 0.0)
                    wbuf[pl.ds(v * 16, 16)] = w
                    vdl = vd - lo
                    cdstl[pl.ds(v * 16, 16)] = vdl
                    pos0 = (v * 16 + lanes) * F
                    for k in range(F):
                        plsc.store_scatter(srcsub, [pos0 + k], vs * F + k)
                        plsc.store_scatter(dstsub, [pos0 + k], vdl * F + k)
                    return 0
                lax.fori_loop(0, nb * (RB // 16), wloop, 0)

                def issue_g(rb, b):
                    pltpu.async_copy(
                        h_hbm.at[srcsub.at[pl.ds(rb * RBF, RBF)]],
                        rowbufs[b], gsems[b])

                for b in range(2):
                    @pl.when(b < nb)
                    def _(b=b):
                        issue_g(jnp.int32(b), b)

                def ring(p, _):
                    for b in range(2):
                        rb = p * 2 + b

                        @pl.when(rb < nb)
                        def _(rb=rb, b=b):
                            buf = rowbufs[b]
                            pltpu.make_async_copy(
                                h_hbm.at[srcsub.at[pl.ds(rb * RBF, RBF)]],
                                buf, gsems[b]).wait()

                            def cpy(k2, _):
                                idxsubs[b][pl.ds(k2 * 16, 16)] = (
                                    dstsub[pl.ds(rb * RBF + k2 * 16, 16)])
                                return 0
                            lax.fori_loop(0, RBF // 16, cpy, 0)

                            def cpy2(k2, _):
                                idxstages[b][pl.ds(k2 * 16, 16)] = (
                                    cdstl[pl.ds(rb * RB + k2 * 16, 16)])
                                wstages[b][pl.ds(k2 * 16, 16)] = (
                                    wbuf[pl.ds(rb * RB + k2 * 16, 16)])
                                return 0
                            lax.fori_loop(0, RB // 16, cpy2, 0)

                            def scale(r, _):
                                wv = plsc.load_gather(
                                    wstages[b],
                                    [jnp.full((16,), r // F, jnp.int32)])
                                for g in range(8):
                                    buf[r, pl.ds(g * 16, 16)] = (
                                        buf[r, pl.ds(g * 16, 16)] * wv)
                                return 0
                            lax.fori_loop(0, RBF, scale, 0, unroll=4)

                            pltpu.sync_copy(buf, numer_s.at[idxsubs[b]],
                                            add=True)
                            pltpu.sync_copy(wstages[b],
                                            denom_s.at[idxstages[b]],
                                            add=True)

                            @pl.when(rb + 2 < nb)
                            def _():
                                issue_g(rb + 2, b)
                    return 0
                lax.fori_loop(0, (nb + 1) // 2, ring, 0)
                return 0

            def bpair(p2, _):
                for pb in range(2):
                    blk(p2 * 2 + pb, pb)
                return 0
            lax.fori_loop(0, NBLK // 2, bpair, 0)
            plsc.subcore_barrier()

            def wb(t, _):
                r0 = s * PERT * F + t * 32
                pltpu.sync_copy(numer_s.at[pl.ds(r0, 32)],
                                rowbuf.at[pl.ds(0, 32)])
                pltpu.sync_copy(rowbuf.at[pl.ds(0, 32)],
                                numer_hbm.at[pl.ds(lo * F + r0, 32)])
                return 0
            lax.fori_loop(0, PERT * F // 32, wb, 0)
            pltpu.sync_copy(denom_s.at[pl.ds(s * PERT, PERT)],
                            dbuf.at[pl.ds(0, PERT)])
            pltpu.sync_copy(dbuf.at[pl.ds(0, PERT)],
                            denom_hbm.at[pl.ds(lo + s * PERT, PERT)])
            plsc.subcore_barrier()

    return edge_kernel


def _make_pool_mlp():
    def body(numer_ref, denom_ref, b_ref, pb_ref, w1_ref, b1_ref,
             w2_ref, b2_ref, out_ref, pooled, counts):
        i = pl.program_id(0)

        @pl.when(i == 0)
        def _():
            pooled[...] = jnp.zeros_like(pooled)
            counts[...] = jnp.zeros_like(counts)

        den = denom_ref[...]
        x = jnp.where(den > 0.0, numer_ref[...] / den, 0.0) + b_ref[...]
        x = jnp.maximum(x, 0.0)
        pb = pb_ref[0, 0, :]
        oh = (pb[None, :] == lax.broadcasted_iota(jnp.int32, (NG, BM), 0)
              ).astype(jnp.float32)
        pooled[...] += jnp.dot(oh, x, preferred_element_type=jnp.float32)
        counts[...] += jnp.sum(oh, axis=1, keepdims=True)

        @pl.when(i == NP // BM - 1)
        def _():
            gc = pooled[...] / jnp.maximum(counts[...], 1.0)
            z = jnp.dot(gc, w1_ref[...], preferred_element_type=jnp.float32)
            z = jnp.maximum(z + b1_ref[...], 0.0)
            out_ref[...] = (jnp.dot(z, w2_ref[...],
                                    preferred_element_type=jnp.float32)
                            + b2_ref[...])

    return pl.pallas_call(
        body,
        grid=(NP // BM,),
        in_specs=[
            pl.BlockSpec((BM, 4 * 128), lambda i: (i, 0)),
            pl.BlockSpec((BM, 1), lambda i: (i, 0)),
            pl.BlockSpec((1, 4 * 128), lambda i: (0, 0)),
            pl.BlockSpec((1, 1, BM), lambda i: (i, 0, 0)),
            pl.BlockSpec((4 * 128, HIDDEN), lambda i: (0, 0)),
            pl.BlockSpec((1, HIDDEN), lambda i: (0, 0)),
            pl.BlockSpec((HIDDEN, NOUT), lambda i: (0, 0)),
            pl.BlockSpec((1, NOUT), lambda i: (0, 0)),
        ],
        out_specs=pl.BlockSpec((NG, NOUT), lambda i: (0, 0)),
        out_shape=jax.ShapeDtypeStruct((NG, NOUT), jnp.float32),
        scratch_shapes=[
            pltpu.VMEM((NG, 4 * 128), jnp.float32),
            pltpu.VMEM((NG, 1), jnp.float32),
        ],
    )


_mm1 = _make_matmul(128, 128, False)
_mm2 = _make_matmul(128, 256, True)
_mm3 = _make_matmul(256, 512, True)
_pool_mlp = _make_pool_mlp()

_edge_cache = {}


def _get_edge(d, n_chunk_per_core):
    key = (d, n_chunk_per_core)
    if key not in _edge_cache:
        _edge_cache[key] = _make_edge_kernel(d, n_chunk_per_core)
    return _edge_cache[key]


@jax.jit
def _run(feature, edge_index, protein_batch, W1, a_src1, a_dst1, b1,
         W2, a_src2, a_dst2, b2, W3, a_src3, a_dst3, b3,
         Wfc1, bfc1, Wfc2, bfc2):
    src = edge_index[0].astype(jnp.int32)
    dst = edge_index[1].astype(jnp.int32)
    pb = jnp.pad(protein_batch.astype(jnp.int32), (0, NP - N),
                 constant_values=NG)
    pb3d = pb.reshape(NP // BM, 1, BM)
    x0 = jnp.pad(feature, ((0, NP - N), (0, 0)))
    A1 = jnp.stack([a_src1, a_dst1], axis=1)
    A2 = jnp.stack([a_src2, a_dst2], axis=1)
    A3 = jnp.stack([a_src3, a_dst3], axis=1)

    h1, as1, ad1 = _mm1(x0, W1, A1)
    num1, den1 = _get_edge(128, 1)(src, dst, as1.reshape(NP), ad1.reshape(NP),
                                   h1.reshape(-1, 128))
    h2, as2, ad2 = _mm2(num1.reshape(NP, 128), den1.reshape(NP, 1),
                        b1.reshape(1, -1), W2, A2)
    num2, den2 = _get_edge(256, 2)(src, dst, as2.reshape(NP), ad2.reshape(NP),
                                   h2.reshape(-1, 128))
    h3, as3, ad3 = _mm3(num2.reshape(NP, 256), den2.reshape(NP, 1),
                        b2.reshape(1, -1), W3, A3)
    num3, den3 = _get_edge(512, 4)(src, dst, as3.reshape(NP), ad3.reshape(NP),
                                   h3.reshape(-1, 128))
    return _pool_mlp(num3.reshape(NP, 512), den3.reshape(NP, 1),
                     b3.reshape(1, -1), pb3d,
                     Wfc1, bfc1.reshape(1, -1), Wfc2, bfc2.reshape(1, -1))


def kernel(feature, edge_index, protein_batch, W1, a_src1, a_dst1, b1,
           W2, a_src2, a_dst2, b2, W3, a_src3, a_dst3, b3,
           Wfc1, bfc1, Wfc2, bfc2):
    return _run(feature, edge_index, protein_batch, W1, a_src1, a_dst1, b1,
                W2, a_src2, a_dst2, b2, W3, a_src3, a_dst3, b3,
                Wfc1, bfc1, Wfc2, bfc2)
